# baseline (speedup 1.0000x reference)
import functools
import math

import jax
import jax.numpy as jnp
from jax import lax
from jax.experimental import pallas as pl
from jax.experimental.pallas import tpu as pltpu
from jax.experimental.pallas import tpu_sc as plsc

B, N, C, S, D = 4, 4096, 64, 64, 256
R = B * N
NC, NS = 2, 16
NW = NC * NS
CHUNK = 128

ROWS_PER_W = R // NW
NCH_S = ROWS_PER_W // CHUNK

P = 2
RH = R // P
ROWS_PER_W_H = RH // NW
NCH_G = ROWS_PER_W_H // CHUNK
NIT_G = 2 * NCH_G
NBUF = 3

_CT = (((1,), (1,)), ((), ()))


@functools.cache
def _sc_kernels():
    mesh = plsc.VectorSubcoreMesh(core_axis_name="c", subcore_axis_name="s")

    def make_gather(h):
        @functools.partial(
            pl.kernel,
            out_type=(jax.ShapeDtypeStruct((RH, D), jnp.float32),
                      jax.ShapeDtypeStruct((RH, D), jnp.float32)),
            mesh=mesh,
            scratch_types=(
                [pltpu.VMEM((NCH_G, CHUNK), jnp.int32)]
                + [pltpu.VMEM((CHUNK, D), jnp.float32)] * NBUF
                + [pltpu.SemaphoreType.DMA] * (2 * NBUF)
            ),
        )
        def sc_gather(geo_hbm, pos_hbm, idx_hbm, outg_hbm, outp_hbm,
                      idx_v, *rest):
            bufs = rest[:NBUF]
            gsems = rest[NBUF:2 * NBUF]
            wsems = rest[2 * NBUF:]
            wid = lax.axis_index("s") * NC + lax.axis_index("c")
            base = wid * ROWS_PER_W_H
            b = h * 2 + wid // (NW // 2)
            o0 = (wid % (NW // 2)) * ROWS_PER_W_H
            for ci in range(NCH_G):
                pltpu.sync_copy(idx_hbm.at[b, pl.ds(o0 + ci * CHUNK, CHUNK)],
                                idx_v.at[ci])

            def src(j):
                tbl = geo_hbm if j % 2 == 0 else pos_hbm
                return tbl.at[b].at[idx_v.at[j // 2]]

            def dst(j):
                out = outg_hbm if j % 2 == 0 else outp_hbm
                return out.at[pl.ds(base + (j // 2) * CHUNK, CHUNK)]

            gcopies = [None] * NIT_G
            wcopies = [None] * NIT_G
            waited = [False] * NIT_G

            def start_gather(j):
                gcopies[j] = pltpu.async_copy(src(j), bufs[j % NBUF],
                                              gsems[j % NBUF])

            start_gather(0)
            start_gather(1)
            for j in range(NIT_G):
                gcopies[j].wait()
                wcopies[j] = pltpu.async_copy(bufs[j % NBUF], dst(j),
                                              wsems[j % NBUF])
                nxt = j + 2
                if nxt < NIT_G:
                    if nxt >= NBUF:
                        wcopies[nxt - NBUF].wait()
                        waited[nxt - NBUF] = True
                    start_gather(nxt)
            for j in range(NIT_G):
                if not waited[j]:
                    wcopies[j].wait()

        return sc_gather

    @functools.partial(
        pl.kernel,
        out_type=jax.ShapeDtypeStruct((B, N, D), jnp.float32),
        mesh=mesh,
        scratch_types=(
            [pltpu.VMEM((NCH_S, CHUNK), jnp.int32)]
            + [pltpu.VMEM((CHUNK, D), jnp.float32)] * 2
            + [pltpu.SemaphoreType.DMA] * 4
        ),
    )
    def sc_scatter(y0_hbm, y1_hbm, idx_hbm, out_hbm, idx_v, buf0, buf1,
                   lsem0, lsem1, ssem0, ssem1):
        bufs = (buf0, buf1)
        lsems = (lsem0, lsem1)
        ssems = (ssem0, ssem1)
        wid = lax.axis_index("s") * NC + lax.axis_index("c")
        b = wid // (N // ROWS_PER_W)
        o0 = (wid % (N // ROWS_PER_W)) * ROWS_PER_W
        for ci in range(NCH_S):
            pltpu.sync_copy(idx_hbm.at[b, pl.ds(o0 + ci * CHUNK, CHUNK)],
                            idx_v.at[ci])

        def run(y_hbm, base):
            loads = [None] * NCH_S
            scats = [None] * NCH_S
            waited = [False] * NCH_S

            def start_load(ci):
                loads[ci] = pltpu.async_copy(
                    y_hbm.at[pl.ds(base + ci * CHUNK, CHUNK)],
                    bufs[ci % 2], lsems[ci % 2])

            start_load(0)
            start_load(1)
            for ci in range(NCH_S):
                loads[ci].wait()
                scats[ci] = pltpu.async_copy(
                    bufs[ci % 2],
                    out_hbm.at[b].at[idx_v.at[ci]],
                    ssems[ci % 2])
                nxt = ci + 2
                if nxt < NCH_S:
                    scats[nxt - 2].wait()
                    waited[nxt - 2] = True
                    start_load(nxt)
            for ci in range(NCH_S):
                if not waited[ci]:
                    scats[ci].wait()

        half = NW // 2

        @pl.when(wid < half)
        def _():
            run(y0_hbm, wid * ROWS_PER_W)

        @pl.when(wid >= half)
        def _():
            run(y1_hbm, wid * ROWS_PER_W - RH)

    return make_gather(0), make_gather(1), sc_scatter


def _fold_body(wo_ref, wv_ref, wq_ref, bv_ref, bo_ref, bq_ref,
               wvo_ref, wqs_ref, bf_ref, bqs_ref):
    scale = 1.0 / math.sqrt(D)
    wvo_ref[...] = lax.dot_general(wo_ref[...], wv_ref[...],
                                   (((1,), (0,)), ((), ())),
                                   preferred_element_type=jnp.float32)
    wqs_ref[...] = wq_ref[...] * scale
    bf_ref[...] = lax.dot_general(bv_ref[...], wo_ref[...], _CT,
                                  preferred_element_type=jnp.float32) + bo_ref[...]
    bqs_ref[...] = bq_ref[...] * scale


_fold_call = pl.pallas_call(
    _fold_body,
    out_shape=(jax.ShapeDtypeStruct((D, D), jnp.float32),
               jax.ShapeDtypeStruct((D, D), jnp.float32),
               jax.ShapeDtypeStruct((1, D), jnp.float32),
               jax.ShapeDtypeStruct((1, D), jnp.float32)),
)

G = 16
ROWS_BLK = G * S
N_BLK = RH // ROWS_BLK


def _attn_body(xg_ref, xp_ref, wqs_ref, wk_ref, wvo_ref, bqs_ref, bf_ref,
               y_ref):
    xg = xg_ref[...]
    xp = xp_ref[...]
    q = lax.dot_general(xg, wqs_ref[...], _CT,
                        preferred_element_type=jnp.float32) + bqs_ref[...]
    k = lax.dot_general(xg, wk_ref[...], _CT,
                        preferred_element_type=jnp.float32)
    v = lax.dot_general(xp, wvo_ref[...], _CT,
                        preferred_element_type=jnp.float32) + bf_ref[...]
    for g in range(G):
        qg = q[g * S:(g + 1) * S]
        kg = k[g * S:(g + 1) * S]
        vg = v[g * S:(g + 1) * S]
        logits = lax.dot_general(qg, kg, _CT,
                                 preferred_element_type=jnp.float32)
        p = jnp.exp(logits)
        attn = p / jnp.sum(p, axis=1, keepdims=True)
        y_ref[pl.ds(g * S, S), :] = jnp.dot(attn, vg,
                                            preferred_element_type=jnp.float32)


_attn_call = pl.pallas_call(
    _attn_body,
    grid=(N_BLK,),
    in_specs=[
        pl.BlockSpec((ROWS_BLK, D), lambda i: (i, 0)),
        pl.BlockSpec((ROWS_BLK, D), lambda i: (i, 0)),
        pl.BlockSpec((D, D), lambda i: (0, 0)),
        pl.BlockSpec((D, D), lambda i: (0, 0)),
        pl.BlockSpec((D, D), lambda i: (0, 0)),
        pl.BlockSpec((1, D), lambda i: (0, 0)),
        pl.BlockSpec((1, D), lambda i: (0, 0)),
    ],
    out_specs=pl.BlockSpec((ROWS_BLK, D), lambda i: (i, 0)),
    out_shape=jax.ShapeDtypeStruct((RH, D), jnp.float32),
    compiler_params=pltpu.CompilerParams(
        dimension_semantics=("parallel",)),
)


def kernel(h_pos, h_geo, kmeans_idx, Wq, bq, Wk, bk, Wv, bv, Wo, bo):
    idx2 = kmeans_idx.reshape(B, N)
    sc_gather0, sc_gather1, sc_scatter = _sc_kernels()
    xg0, xp0 = sc_gather0(h_geo, h_pos, idx2)
    xg1, xp1 = sc_gather1(h_geo, h_pos, idx2)
    wvo, wqs, bfold, bqs = _fold_call(Wo, Wv, Wq, bv[None, :], bo[None, :],
                                      bq[None, :])
    y0 = _attn_call(xg0, xp0, wqs, Wk, wvo, bqs, bfold)
    y1 = _attn_call(xg1, xp1, wqs, Wk, wvo, bqs, bfold)
    return sc_scatter(y0, y1, idx2)


# speedup vs baseline: 1.0185x; 1.0185x over previous
import functools
import math

import jax
import jax.numpy as jnp
from jax import lax
from jax.experimental import pallas as pl
from jax.experimental.pallas import tpu as pltpu
from jax.experimental.pallas import tpu_sc as plsc

B, N, C, S, D = 4, 4096, 64, 64, 256
R = B * N
NC, NS = 2, 16
NW = NC * NS
CHUNK = 128

ROWS_PER_W = R // NW
NCHUNK = ROWS_PER_W // CHUNK

P = 2
RH = R // P
ROWS_PER_W_H = RH // NW
NCHUNK_H = ROWS_PER_W_H // CHUNK
NIT_H = 2 * NCHUNK_H
NBUF = 3

_CT = (((1,), (1,)), ((), ()))


@functools.cache
def _sc_kernels():
    mesh = plsc.VectorSubcoreMesh(core_axis_name="c", subcore_axis_name="s")

    def make_gather(h):
        @functools.partial(
            pl.kernel,
            out_type=(jax.ShapeDtypeStruct((RH, D), jnp.float32),
                      jax.ShapeDtypeStruct((RH, D), jnp.float32)),
            mesh=mesh,
            scratch_types=(
                [pltpu.VMEM((NCHUNK_H, CHUNK), jnp.int32)]
                + [pltpu.VMEM((CHUNK, D), jnp.float32)] * NBUF
                + [pltpu.SemaphoreType.DMA] * (2 * NBUF)
            ),
        )
        def sc_gather(geo_hbm, pos_hbm, idx3_hbm, outg_hbm, outp_hbm,
                      idx_v, *rest):
            bufs = rest[:NBUF]
            gsems = rest[NBUF:2 * NBUF]
            wsems = rest[2 * NBUF:]
            wid = lax.axis_index("s") * NC + lax.axis_index("c")
            base = wid * ROWS_PER_W_H
            boff = (h * 2 + wid // (NW // 2)) * N
            gw = h * (NW // 2) + wid // 2
            pltpu.sync_copy(
                idx3_hbm.at[gw, pl.ds((wid % 2) * NCHUNK_H, NCHUNK_H)],
                idx_v)

            def src(j):
                tbl = geo_hbm if j % 2 == 0 else pos_hbm
                return tbl.at[pl.ds(boff, N)].at[idx_v.at[j // 2]]

            def dst(j):
                out = outg_hbm if j % 2 == 0 else outp_hbm
                return out.at[pl.ds(base + (j // 2) * CHUNK, CHUNK)]

            gcopies = [None] * NIT_H
            wcopies = [None] * NIT_H
            waited = [False] * NIT_H

            def start_gather(j):
                gcopies[j] = pltpu.async_copy(src(j), bufs[j % NBUF],
                                              gsems[j % NBUF])

            start_gather(0)
            start_gather(1)
            for j in range(NIT_H):
                gcopies[j].wait()
                wcopies[j] = pltpu.async_copy(bufs[j % NBUF], dst(j),
                                              wsems[j % NBUF])
                nxt = j + 2
                if nxt < NIT_H:
                    if nxt >= NBUF:
                        wcopies[nxt - NBUF].wait()
                        waited[nxt - NBUF] = True
                    start_gather(nxt)
            for j in range(NIT_H):
                if not waited[j]:
                    wcopies[j].wait()

        return sc_gather

    @functools.partial(
        pl.kernel,
        out_type=jax.ShapeDtypeStruct((R, D), jnp.float32),
        mesh=mesh,
        scratch_types=(
            [pltpu.VMEM((NCHUNK, CHUNK), jnp.int32)]
            + [pltpu.VMEM((CHUNK, D), jnp.float32)] * 2
            + [pltpu.SemaphoreType.DMA] * 4
        ),
    )
    def sc_scatter(y0_hbm, y1_hbm, idx3_hbm, out_hbm, idx_v, buf0, buf1,
                   lsem0, lsem1, ssem0, ssem1):
        bufs = (buf0, buf1)
        lsems = (lsem0, lsem1)
        ssems = (ssem0, ssem1)
        wid = lax.axis_index("s") * NC + lax.axis_index("c")
        boff = (wid // (N // ROWS_PER_W)) * N
        pltpu.sync_copy(idx3_hbm.at[wid], idx_v)

        def run(y_hbm, base):
            loads = [None] * NCHUNK
            scats = [None] * NCHUNK
            waited = [False] * NCHUNK

            def start_load(ci):
                loads[ci] = pltpu.async_copy(
                    y_hbm.at[pl.ds(base + ci * CHUNK, CHUNK)],
                    bufs[ci % 2], lsems[ci % 2])

            start_load(0)
            start_load(1)
            for ci in range(NCHUNK):
                loads[ci].wait()
                scats[ci] = pltpu.async_copy(
                    bufs[ci % 2],
                    out_hbm.at[pl.ds(boff, N)].at[idx_v.at[ci]],
                    ssems[ci % 2])
                nxt = ci + 2
                if nxt < NCHUNK:
                    scats[nxt - 2].wait()
                    waited[nxt - 2] = True
                    start_load(nxt)
            for ci in range(NCHUNK):
                if not waited[ci]:
                    scats[ci].wait()

        half = NW // 2

        @pl.when(wid < half)
        def _():
            run(y0_hbm, wid * ROWS_PER_W)

        @pl.when(wid >= half)
        def _():
            run(y1_hbm, wid * ROWS_PER_W - RH)

    return make_gather(0), make_gather(1), sc_scatter


def _fold_body(wo_ref, wv_ref, wq_ref, wk_ref, bv_ref, bo_ref, bq_ref,
               wvob_ref, wqb_ref, wkb_ref, bf_ref, bqs_ref):
    scale = 1.0 / math.sqrt(D)
    wvo = lax.dot_general(wo_ref[...], wv_ref[...],
                          (((1,), (0,)), ((), ())),
                          preferred_element_type=jnp.float32)
    wvob_ref[...] = wvo.astype(jnp.bfloat16)
    wqb_ref[...] = (wq_ref[...] * scale).astype(jnp.bfloat16)
    wkb_ref[...] = wk_ref[...].astype(jnp.bfloat16)
    bf_ref[...] = lax.dot_general(bv_ref[...], wo_ref[...], _CT,
                                  preferred_element_type=jnp.float32) + bo_ref[...]
    bqs_ref[...] = bq_ref[...] * scale


_fold_call = pl.pallas_call(
    _fold_body,
    out_shape=(jax.ShapeDtypeStruct((D, D), jnp.bfloat16),
               jax.ShapeDtypeStruct((D, D), jnp.bfloat16),
               jax.ShapeDtypeStruct((D, D), jnp.bfloat16),
               jax.ShapeDtypeStruct((1, D), jnp.float32),
               jax.ShapeDtypeStruct((1, D), jnp.float32)),
)

G = 16
ROWS_BLK = G * S
N_BLK = RH // ROWS_BLK


def _attn_body(xg_ref, xp_ref, wqb_ref, wkb_ref, wvob_ref, bqs_ref, bf_ref,
               y_ref):
    xgb = xg_ref[...].astype(jnp.bfloat16)
    xpb = xp_ref[...].astype(jnp.bfloat16)
    q = lax.dot_general(xgb, wqb_ref[...], _CT,
                        preferred_element_type=jnp.float32) + bqs_ref[...]
    k = lax.dot_general(xgb, wkb_ref[...], _CT,
                        preferred_element_type=jnp.float32)
    v = lax.dot_general(xpb, wvob_ref[...], _CT,
                        preferred_element_type=jnp.float32) + bf_ref[...]
    for g in range(G):
        qg = q[g * S:(g + 1) * S]
        kg = k[g * S:(g + 1) * S]
        vg = v[g * S:(g + 1) * S]
        logits = lax.dot_general(qg, kg, _CT,
                                 preferred_element_type=jnp.float32)
        p = jnp.exp(logits)
        attn = p / jnp.sum(p, axis=1, keepdims=True)
        y_ref[pl.ds(g * S, S), :] = jnp.dot(attn, vg,
                                            preferred_element_type=jnp.float32)


_attn_call = pl.pallas_call(
    _attn_body,
    grid=(N_BLK,),
    in_specs=[
        pl.BlockSpec((ROWS_BLK, D), lambda i: (i, 0)),
        pl.BlockSpec((ROWS_BLK, D), lambda i: (i, 0)),
        pl.BlockSpec((D, D), lambda i: (0, 0)),
        pl.BlockSpec((D, D), lambda i: (0, 0)),
        pl.BlockSpec((D, D), lambda i: (0, 0)),
        pl.BlockSpec((1, D), lambda i: (0, 0)),
        pl.BlockSpec((1, D), lambda i: (0, 0)),
    ],
    out_specs=pl.BlockSpec((ROWS_BLK, D), lambda i: (i, 0)),
    out_shape=jax.ShapeDtypeStruct((RH, D), jnp.float32),
    compiler_params=pltpu.CompilerParams(
        dimension_semantics=("parallel",)),
)


def kernel(h_pos, h_geo, kmeans_idx, Wq, bq, Wk, bk, Wv, bv, Wo, bo):
    geo = h_geo.reshape(R, D)
    pos = h_pos.reshape(R, D)
    idx3 = kmeans_idx.reshape(NW, NCHUNK, CHUNK)

    sc_gather0, sc_gather1, sc_scatter = _sc_kernels()
    xg0, xp0 = sc_gather0(geo, pos, idx3)
    xg1, xp1 = sc_gather1(geo, pos, idx3)
    wvob, wqb, wkb, bfold, bqs = _fold_call(Wo, Wv, Wq, Wk, bv[None, :],
                                            bo[None, :], bq[None, :])
    y0 = _attn_call(xg0, xp0, wqb, wkb, wvob, bqs, bfold)
    y1 = _attn_call(xg1, xp1, wqb, wkb, wvob, bqs, bfold)
    out = sc_scatter(y0, y1, idx3)
    return out.reshape(B, N, D)


# speedup vs baseline: 1.0407x; 1.0219x over previous
import functools
import math

import jax
import jax.numpy as jnp
from jax import lax
from jax.experimental import pallas as pl
from jax.experimental.pallas import tpu as pltpu
from jax.experimental.pallas import tpu_sc as plsc

B, N, C, S, D = 4, 4096, 64, 64, 256
R = B * N
NC, NS = 2, 16
NW = NC * NS
CHUNK = 128

ROWS_PER_W = R // NW
NCHUNK = ROWS_PER_W // CHUNK

P = 2
RH = R // P
ROWS_PER_W_H = RH // NW
NCHUNK_H = ROWS_PER_W_H // CHUNK
NIT_H = 2 * NCHUNK_H
NBUF = 3

_CT = (((1,), (1,)), ((), ()))


@functools.cache
def _sc_kernels():
    mesh = plsc.VectorSubcoreMesh(core_axis_name="c", subcore_axis_name="s")

    def make_gather(h):
        @functools.partial(
            pl.kernel,
            out_type=(jax.ShapeDtypeStruct((RH, D), jnp.float32),
                      jax.ShapeDtypeStruct((RH, D), jnp.float32)),
            mesh=mesh,
            scratch_types=(
                [pltpu.VMEM((NCHUNK_H, CHUNK), jnp.int32)]
                + [pltpu.VMEM((CHUNK, D), jnp.float32)] * NBUF
                + [pltpu.SemaphoreType.DMA] * (2 * NBUF)
            ),
        )
        def sc_gather(geo_hbm, pos_hbm, idx3_hbm, outg_hbm, outp_hbm,
                      idx_v, *rest):
            bufs = rest[:NBUF]
            gsems = rest[NBUF:2 * NBUF]
            wsems = rest[2 * NBUF:]
            wid = lax.axis_index("s") * NC + lax.axis_index("c")
            base = wid * ROWS_PER_W_H
            boff = (h * 2 + wid // (NW // 2)) * N
            gw = h * (NW // 2) + wid // 2
            pltpu.sync_copy(
                idx3_hbm.at[gw, pl.ds((wid % 2) * NCHUNK_H, NCHUNK_H)],
                idx_v)

            def src(j):
                tbl = geo_hbm if j % 2 == 0 else pos_hbm
                return tbl.at[pl.ds(boff, N)].at[idx_v.at[j // 2]]

            def dst(j):
                out = outg_hbm if j % 2 == 0 else outp_hbm
                return out.at[pl.ds(base + (j // 2) * CHUNK, CHUNK)]

            gcopies = [None] * NIT_H
            wcopies = [None] * NIT_H
            waited = [False] * NIT_H

            def start_gather(j):
                gcopies[j] = pltpu.async_copy(src(j), bufs[j % NBUF],
                                              gsems[j % NBUF])

            start_gather(0)
            start_gather(1)
            for j in range(NIT_H):
                gcopies[j].wait()
                wcopies[j] = pltpu.async_copy(bufs[j % NBUF], dst(j),
                                              wsems[j % NBUF])
                nxt = j + 2
                if nxt < NIT_H:
                    if nxt >= NBUF:
                        wcopies[nxt - NBUF].wait()
                        waited[nxt - NBUF] = True
                    start_gather(nxt)
            for j in range(NIT_H):
                if not waited[j]:
                    wcopies[j].wait()

        return sc_gather

    @functools.partial(
        pl.kernel,
        out_type=jax.ShapeDtypeStruct((R, D), jnp.float32),
        mesh=mesh,
        scratch_types=(
            [pltpu.VMEM((NCHUNK, CHUNK), jnp.int32)]
            + [pltpu.VMEM((CHUNK, D), jnp.float32)] * 2
            + [pltpu.SemaphoreType.DMA] * 4
        ),
    )
    def sc_scatter(y0_hbm, y1_hbm, idx3_hbm, out_hbm, idx_v, buf0, buf1,
                   lsem0, lsem1, ssem0, ssem1):
        bufs = (buf0, buf1)
        lsems = (lsem0, lsem1)
        ssems = (ssem0, ssem1)
        wid = lax.axis_index("s") * NC + lax.axis_index("c")
        boff = (wid // (N // ROWS_PER_W)) * N
        pltpu.sync_copy(idx3_hbm.at[wid], idx_v)

        def run(y_hbm, base):
            loads = [None] * NCHUNK
            scats = [None] * NCHUNK
            waited = [False] * NCHUNK

            def start_load(ci):
                loads[ci] = pltpu.async_copy(
                    y_hbm.at[pl.ds(base + ci * CHUNK, CHUNK)],
                    bufs[ci % 2], lsems[ci % 2])

            start_load(0)
            start_load(1)
            for ci in range(NCHUNK):
                loads[ci].wait()
                scats[ci] = pltpu.async_copy(
                    bufs[ci % 2],
                    out_hbm.at[pl.ds(boff, N)].at[idx_v.at[ci]],
                    ssems[ci % 2])
                nxt = ci + 2
                if nxt < NCHUNK:
                    scats[nxt - 2].wait()
                    waited[nxt - 2] = True
                    start_load(nxt)
            for ci in range(NCHUNK):
                if not waited[ci]:
                    scats[ci].wait()

        half = NW // 2

        @pl.when(wid < half)
        def _():
            run(y0_hbm, wid * ROWS_PER_W)

        @pl.when(wid >= half)
        def _():
            run(y1_hbm, wid * ROWS_PER_W - RH)

    return make_gather(0), make_gather(1), sc_scatter


def _fold_body(wo_ref, wv_ref, wq_ref, wk_ref, bv_ref, bo_ref, bq_ref,
               wvob_ref, wqb_ref, wkb_ref, bf_ref, bqs_ref):
    scale = 1.0 / math.sqrt(D)
    wvo = lax.dot_general(wo_ref[...], wv_ref[...],
                          (((1,), (0,)), ((), ())),
                          preferred_element_type=jnp.float32)
    wvob_ref[...] = wvo.astype(jnp.bfloat16)
    wqb_ref[...] = (wq_ref[...] * scale).astype(jnp.bfloat16)
    wkb_ref[...] = wk_ref[...].astype(jnp.bfloat16)
    bf_ref[...] = lax.dot_general(bv_ref[...], wo_ref[...], _CT,
                                  preferred_element_type=jnp.float32) + bo_ref[...]
    bqs_ref[...] = bq_ref[...] * scale


_fold_call = pl.pallas_call(
    _fold_body,
    out_shape=(jax.ShapeDtypeStruct((D, D), jnp.bfloat16),
               jax.ShapeDtypeStruct((D, D), jnp.bfloat16),
               jax.ShapeDtypeStruct((D, D), jnp.bfloat16),
               jax.ShapeDtypeStruct((1, D), jnp.float32),
               jax.ShapeDtypeStruct((1, D), jnp.float32)),
)

G = 32
ROWS_BLK = G * S
N_BLK = RH // ROWS_BLK


def _attn_body(xg_ref, xp_ref, wqb_ref, wkb_ref, wvob_ref, bqs_ref, bf_ref,
               y_ref):
    xgb = xg_ref[...].astype(jnp.bfloat16)
    xpb = xp_ref[...].astype(jnp.bfloat16)
    q = lax.dot_general(xgb, wqb_ref[...], _CT,
                        preferred_element_type=jnp.float32) + bqs_ref[...]
    k = lax.dot_general(xgb, wkb_ref[...], _CT,
                        preferred_element_type=jnp.float32)
    v = lax.dot_general(xpb, wvob_ref[...], _CT,
                        preferred_element_type=jnp.float32) + bf_ref[...]
    for g in range(G):
        qg = q[g * S:(g + 1) * S]
        kg = k[g * S:(g + 1) * S]
        vg = v[g * S:(g + 1) * S]
        logits = lax.dot_general(qg, kg, _CT,
                                 preferred_element_type=jnp.float32)
        p = jnp.exp(logits)
        attn = p / jnp.sum(p, axis=1, keepdims=True)
        y_ref[pl.ds(g * S, S), :] = jnp.dot(attn, vg,
                                            preferred_element_type=jnp.float32)


_attn_call = pl.pallas_call(
    _attn_body,
    grid=(N_BLK,),
    in_specs=[
        pl.BlockSpec((ROWS_BLK, D), lambda i: (i, 0)),
        pl.BlockSpec((ROWS_BLK, D), lambda i: (i, 0)),
        pl.BlockSpec((D, D), lambda i: (0, 0)),
        pl.BlockSpec((D, D), lambda i: (0, 0)),
        pl.BlockSpec((D, D), lambda i: (0, 0)),
        pl.BlockSpec((1, D), lambda i: (0, 0)),
        pl.BlockSpec((1, D), lambda i: (0, 0)),
    ],
    out_specs=pl.BlockSpec((ROWS_BLK, D), lambda i: (i, 0)),
    out_shape=jax.ShapeDtypeStruct((RH, D), jnp.float32),
    compiler_params=pltpu.CompilerParams(
        dimension_semantics=("parallel",)),
)


def kernel(h_pos, h_geo, kmeans_idx, Wq, bq, Wk, bk, Wv, bv, Wo, bo):
    geo = h_geo.reshape(R, D)
    pos = h_pos.reshape(R, D)
    idx3 = kmeans_idx.reshape(NW, NCHUNK, CHUNK)

    sc_gather0, sc_gather1, sc_scatter = _sc_kernels()
    xg0, xp0 = sc_gather0(geo, pos, idx3)
    xg1, xp1 = sc_gather1(geo, pos, idx3)
    wvob, wqb, wkb, bfold, bqs = _fold_call(Wo, Wv, Wq, Wk, bv[None, :],
                                            bo[None, :], bq[None, :])
    y0 = _attn_call(xg0, xp0, wqb, wkb, wvob, bqs, bfold)
    y1 = _attn_call(xg1, xp1, wqb, wkb, wvob, bqs, bfold)
    out = sc_scatter(y0, y1, idx3)
    return out.reshape(B, N, D)
